# baseline (speedup 1.0000x reference)
import functools

import jax
import jax.numpy as jnp
from jax import lax
from jax.experimental import pallas as pl
from jax.experimental.pallas import tpu as pltpu
from jax.experimental.pallas import tpu_sc as plsc

NC = 2
NS = 16
LANES = 128
NWORK = NC * NS


def _sc_laplacian(vmat, epack, evala, evalb, *, TR, VPAD, EW):
  B = vmat.shape[0]
  FPAD = 3 * VPAD
  QP = VPAD // 4
  assert TR * LANES == FPAD and QP % LANES == 0

  mesh = plsc.VectorSubcoreMesh(
      core_axis_name="c", subcore_axis_name="s", num_cores=NC,
      num_subcores=NS)

  @functools.partial(
      pl.kernel,
      out_type=jax.ShapeDtypeStruct((NWORK * 16,), jnp.float32),
      mesh=mesh,
      compiler_params=pltpu.CompilerParams(needs_layout_passes=False),
      scratch_types=[
          pltpu.VMEM((TR, LANES), jnp.float32),
          pltpu.VMEM((FPAD,), jnp.float32),
          pltpu.VMEM((EW,), jnp.int32),
          pltpu.VMEM((EW,), jnp.float32),
          pltpu.VMEM((EW,), jnp.float32),
          pltpu.VMEM((16,), jnp.float32),
          pltpu.VMEM_SHARED((B // NC, 4, FPAD), jnp.float32),
          pltpu.SemaphoreType.DMA,
      ],
  )
  def k(vmat_hbm, pack_hbm, vala_hbm, valb_hbm, out_hbm,
        vpl, agg, epk, eva, evb, outv, acc_sh, sem):
    cid = lax.axis_index("c")
    sid = lax.axis_index("s")
    wid = cid * NS + sid
    lb = sid // 4
    b = cid * (B // NC) + lb
    w = sid % 4

    d1 = pltpu.async_copy(vmat_hbm.at[b], vpl, sem)
    d2 = pltpu.async_copy(pack_hbm.at[pl.ds(w * EW, EW)], epk, sem)
    d3 = pltpu.async_copy(vala_hbm.at[pl.ds(w * EW, EW)], eva, sem)
    d4 = pltpu.async_copy(valb_hbm.at[pl.ds(w * EW, EW)], evb, sem)

    zero16 = jnp.zeros((16,), jnp.float32)

    @plsc.parallel_loop(0, FPAD // 16, step=1, unroll=8)
    def zbody(i):
      agg[pl.ds(i * 16, 16)] = zero16

    d1.wait()
    d2.wait()
    d3.wait()
    d4.wait()

    @plsc.parallel_loop(0, EW, step=16, unroll=4)
    def ebody(o):
      p = epk[pl.ds(o, 16)]
      wa = eva[pl.ds(o, 16)]
      wb = evb[pl.ds(o, 16)]
      ea = p >> 15
      eb = p & 32767
      for cc in range(3):
        fa = ea + cc * VPAD
        fb = eb + cc * VPAD
        ga = plsc.load_gather(vpl, [fa >> 7, fa & 127])
        gb = plsc.load_gather(vpl, [fb >> 7, fb & 127])
        plsc.addupdate_scatter(agg, [fa], wa * gb)
        plsc.addupdate_scatter(agg, [fb], wb * ga)

    pltpu.sync_copy(agg, acc_sh.at[lb, w])

    plsc.subcore_barrier()

    ds = [pltpu.async_copy(
        acc_sh.at[lb, j, pl.ds(cc * VPAD + w * QP, QP)],
        agg.at[pl.ds((j * 3 + cc) * QP, QP)], sem)
        for j in range(4) for cc in range(3)]
    for d in ds:
      d.wait()

    total = jnp.zeros((16,), jnp.float32)
    for cc in range(3):
      qrow = cc * (VPAD // LANES) + w * (QP // LANES)

      @plsc.parallel_loop(0, QP // 16, step=1, unroll=4,
                          carry=jnp.zeros((16,), jnp.float32))
      def rbody(i, acc):
        o = i * 16
        x = (vpl[qrow + (i >> 3), pl.ds((i & 7) * 16, 16)]
             + agg[pl.ds((0 * 3 + cc) * QP + o, 16)]
             + agg[pl.ds((1 * 3 + cc) * QP + o, 16)]
             + agg[pl.ds((2 * 3 + cc) * QP + o, 16)]
             + agg[pl.ds((3 * 3 + cc) * QP + o, 16)])
        return acc + x * x

      total = total + rbody

    outv[...] = total
    pltpu.sync_copy(outv, out_hbm.at[pl.ds(wid * 16, 16)])

  return k(vmat, epack, evala, evalb)


def kernel(vertices, rows, cols, vals):
  if vertices.ndim == 2:
    vertices = vertices[None]
  B, V, C = vertices.shape
  E = rows.shape[0]
  H = E // 2
  assert V <= 32768 and E % 2 == 0

  VPAD = ((V + 511) // 512) * 512
  TR = 3 * (VPAD // LANES)
  HPAD = ((H + 255) // 256) * 256
  EW = HPAD // 4

  vt = jnp.transpose(vertices, (0, 2, 1))
  vt = jnp.pad(vt, ((0, 0), (0, 0), (0, VPAD - V)))
  vmat = vt.reshape(B, TR, LANES)

  pe = HPAD - H
  a = rows[:H].astype(jnp.int32)
  bb = cols[:H].astype(jnp.int32)
  epack = jnp.pad(a * 32768 + bb, (0, pe))
  evala = jnp.pad(vals[:H].astype(jnp.float32), (0, pe))
  evalb = jnp.pad(vals[H:].astype(jnp.float32), (0, pe))

  out = _sc_laplacian(vmat, epack, evala, evalb, TR=TR, VPAD=VPAD, EW=EW)
  return jnp.sum(out) / (B * V)


# speedup vs baseline: 1.0568x; 1.0568x over previous
import functools

import jax
import jax.numpy as jnp
from jax import lax
from jax.experimental import pallas as pl
from jax.experimental.pallas import tpu as pltpu
from jax.experimental.pallas import tpu_sc as plsc

NC = 2
NS = 16
LANES = 128
NWORK = NC * NS


def _sc_laplacian(vmat, epack, evala, evalb, *, TR, EW, VPAD):
  B = vmat.shape[0]
  QR = TR // 4
  PUB = TR // 3
  assert TR % 12 == 0 and PUB <= 128

  mesh = plsc.VectorSubcoreMesh(
      core_axis_name="c", subcore_axis_name="s", num_cores=NC,
      num_subcores=NS)

  @functools.partial(
      pl.kernel,
      out_type=jax.ShapeDtypeStruct((NWORK * 16,), jnp.float32),
      mesh=mesh,
      compiler_params=pltpu.CompilerParams(needs_layout_passes=False),
      scratch_types=[
          pltpu.VMEM((TR, LANES), jnp.float32),
          pltpu.VMEM((TR, LANES), jnp.float32),
          pltpu.VMEM((EW,), jnp.int32),
          pltpu.VMEM((EW,), jnp.float32),
          pltpu.VMEM((EW,), jnp.float32),
          pltpu.VMEM((3, PUB), jnp.int32),
          pltpu.VMEM((16,), jnp.float32),
          pltpu.VMEM_SHARED((B // NC, TR, LANES), jnp.float32),
          pltpu.SemaphoreType.DMA,
      ],
  )
  def k(vmat_hbm, pack_hbm, vala_hbm, valb_hbm, out_hbm,
        vpl, agg, epk, eva, evb, idxr, outv, acc_sh, sem):
    cid = lax.axis_index("c")
    sid = lax.axis_index("s")
    wid = cid * NS + sid
    lb = sid // 4
    b = cid * (B // NC) + lb
    w = sid % 4

    d1 = pltpu.async_copy(vmat_hbm.at[b], vpl, sem)
    d2 = pltpu.async_copy(pack_hbm.at[pl.ds(w * EW, EW)], epk, sem)
    d3 = pltpu.async_copy(vala_hbm.at[pl.ds(w * EW, EW)], eva, sem)
    d4 = pltpu.async_copy(valb_hbm.at[pl.ds(w * EW, EW)], evb, sem)

    zero16 = jnp.zeros((16,), jnp.float32)

    @plsc.parallel_loop(0, TR, step=1, unroll=4)
    def zbody(i):
      for kk in range(LANES // 16):
        agg[i, pl.ds(kk * 16, 16)] = zero16

    iota = lax.iota(jnp.int32, 16)
    for j in range(3):
      for kk in range(PUB // 16):
        idxr[j, pl.ds(kk * 16, 16)] = iota + (j * PUB + kk * 16)

    d1.wait()
    d2.wait()
    d3.wait()
    d4.wait()

    @pl.when(w == 0)
    def _():
      pltpu.sync_copy(vpl, acc_sh.at[lb])

    plsc.subcore_barrier()

    @plsc.parallel_loop(0, EW, step=16, unroll=2)
    def ebody(o):
      p = epk[pl.ds(o, 16)]
      wa = eva[pl.ds(o, 16)]
      wb = evb[pl.ds(o, 16)]
      ea = p >> 15
      eb = p & 32767
      for cc in range(3):
        fa = ea + cc * VPAD
        fb = eb + cc * VPAD
        ga = plsc.load_gather(vpl, [fa >> 7, fa & 127])
        gb = plsc.load_gather(vpl, [fb >> 7, fb & 127])
        plsc.addupdate_scatter(agg, [fa >> 7, fa & 127], wa * gb)
        plsc.addupdate_scatter(agg, [fb >> 7, fb & 127], wb * ga)

    pubs = [pltpu.async_copy(agg.at[pl.ds(j * PUB, PUB)],
                             acc_sh.at[lb].at[idxr.at[j]], sem, add=True)
            for j in range(3)]
    for p in pubs:
      p.wait()

    plsc.subcore_barrier()

    pltpu.sync_copy(acc_sh.at[lb].at[pl.ds(w * QR, QR)], vpl.at[pl.ds(0, QR)])

    @plsc.parallel_loop(0, QR * (LANES // 16), step=1, unroll=8,
                        carry=jnp.zeros((16,), jnp.float32))
    def rbody(i, acc):
      r = i >> 3
      co = (i & 7) * 16
      x = vpl[r, pl.ds(co, 16)]
      return acc + x * x

    outv[...] = rbody
    pltpu.sync_copy(outv, out_hbm.at[pl.ds(wid * 16, 16)])

  return k(vmat, epack, evala, evalb)


def kernel(vertices, rows, cols, vals):
  if vertices.ndim == 2:
    vertices = vertices[None]
  B, V, C = vertices.shape
  E = rows.shape[0]
  H = E // 2
  assert V <= 32768 and E % 2 == 0

  VPAD = ((V + 511) // 512) * 512
  TR = 3 * (VPAD // LANES)
  HPAD = ((H + 255) // 256) * 256
  EW = HPAD // 4

  vt = jnp.transpose(vertices, (0, 2, 1))
  vt = jnp.pad(vt, ((0, 0), (0, 0), (0, VPAD - V)))
  vmat = vt.reshape(B, TR, LANES)

  pe = HPAD - H
  a = rows[:H].astype(jnp.int32)
  bb = cols[:H].astype(jnp.int32)
  epack = jnp.pad(a * 32768 + bb, (0, pe))
  evala = jnp.pad(vals[:H].astype(jnp.float32), (0, pe))
  evalb = jnp.pad(vals[H:].astype(jnp.float32), (0, pe))

  out = _sc_laplacian(vmat, epack, evala, evalb, TR=TR, EW=EW, VPAD=VPAD)
  return jnp.sum(out) / (B * V)


# speedup vs baseline: 1.0592x; 1.0022x over previous
import functools

import jax
import jax.numpy as jnp
from jax import lax
from jax.experimental import pallas as pl
from jax.experimental.pallas import tpu as pltpu
from jax.experimental.pallas import tpu_sc as plsc

NC = 2
NS = 16
LANES = 128
NWORK = NC * NS


def _sc_laplacian(vmat, epack, evala, evalb, *, TR, EW, VPAD):
  B = vmat.shape[0]
  QR = TR // 4
  PUB = TR // 3
  assert TR % 12 == 0 and PUB <= 128

  mesh = plsc.VectorSubcoreMesh(
      core_axis_name="c", subcore_axis_name="s", num_cores=NC,
      num_subcores=NS)

  @functools.partial(
      pl.kernel,
      out_type=jax.ShapeDtypeStruct((NWORK * 16,), jnp.float32),
      mesh=mesh,
      compiler_params=pltpu.CompilerParams(needs_layout_passes=False),
      scratch_types=[
          pltpu.VMEM((TR, LANES), jnp.float32),
          pltpu.VMEM((TR, LANES), jnp.float32),
          pltpu.VMEM((EW,), jnp.int32),
          pltpu.VMEM((EW,), jnp.float32),
          pltpu.VMEM((EW,), jnp.float32),
          pltpu.VMEM((3, PUB), jnp.int32),
          pltpu.VMEM((16,), jnp.float32),
          pltpu.VMEM_SHARED((B // NC, TR, LANES), jnp.float32),
          pltpu.SemaphoreType.DMA,
      ],
  )
  def k(vmat_hbm, pack_hbm, vala_hbm, valb_hbm, out_hbm,
        vpl, agg, epk, eva, evb, idxr, outv, acc_sh, sem):
    cid = lax.axis_index("c")
    sid = lax.axis_index("s")
    wid = cid * NS + sid
    lb = sid // 4
    b = cid * (B // NC) + lb
    w = sid % 4

    d1 = pltpu.async_copy(vmat_hbm.at[b], vpl, sem)
    d2 = pltpu.async_copy(pack_hbm.at[pl.ds(w * EW, EW)], epk, sem)
    d3 = pltpu.async_copy(vala_hbm.at[pl.ds(w * EW, EW)], eva, sem)
    d4 = pltpu.async_copy(valb_hbm.at[pl.ds(w * EW, EW)], evb, sem)

    zero16 = jnp.zeros((16,), jnp.float32)

    @plsc.parallel_loop(0, TR, step=1, unroll=2)
    def zbody(i):
      for kk in range(LANES // 16):
        agg[i, pl.ds(kk * 16, 16)] = zero16

    iota = lax.iota(jnp.int32, 16)
    for j in range(3):
      for kk in range(PUB // 16):
        idxr[j, pl.ds(kk * 16, 16)] = iota + (j * PUB + kk * 16)

    d1.wait()
    d2.wait()
    d3.wait()
    d4.wait()

    @pl.when(w == 0)
    def _():
      pltpu.sync_copy(vpl, acc_sh.at[lb])

    plsc.subcore_barrier()

    @plsc.parallel_loop(0, EW, step=16, unroll=2)
    def ebody(o):
      p = epk[pl.ds(o, 16)]
      wa = eva[pl.ds(o, 16)]
      wb = evb[pl.ds(o, 16)]
      ea = p >> 15
      eb = p & 32767
      for cc in range(3):
        fa = ea + cc * VPAD
        fb = eb + cc * VPAD
        ga = plsc.load_gather(vpl, [fa >> 7, fa & 127])
        gb = plsc.load_gather(vpl, [fb >> 7, fb & 127])
        plsc.addupdate_scatter(agg, [fa >> 7, fa & 127], wa * gb)
        plsc.addupdate_scatter(agg, [fb >> 7, fb & 127], wb * ga)

    pubs = [pltpu.async_copy(agg.at[pl.ds(j * PUB, PUB)],
                             acc_sh.at[lb].at[idxr.at[j]], sem, add=True)
            for j in range(3)]
    for p in pubs:
      p.wait()

    plsc.subcore_barrier()

    pltpu.sync_copy(acc_sh.at[lb].at[pl.ds(w * QR, QR)], vpl.at[pl.ds(0, QR)])

    @plsc.parallel_loop(0, QR * (LANES // 16), step=1, unroll=4,
                        carry=jnp.zeros((16,), jnp.float32))
    def rbody(i, acc):
      r = i >> 3
      co = (i & 7) * 16
      x = vpl[r, pl.ds(co, 16)]
      return acc + x * x

    outv[...] = rbody
    pltpu.sync_copy(outv, out_hbm.at[pl.ds(wid * 16, 16)])

  return k(vmat, epack, evala, evalb)


def kernel(vertices, rows, cols, vals):
  if vertices.ndim == 2:
    vertices = vertices[None]
  B, V, C = vertices.shape
  E = rows.shape[0]
  H = E // 2
  assert V <= 32768 and E % 2 == 0

  VPAD = ((V + 511) // 512) * 512
  TR = 3 * (VPAD // LANES)
  HPAD = ((H + 255) // 256) * 256
  EW = HPAD // 4

  vt = jnp.transpose(vertices, (0, 2, 1))
  vt = jnp.pad(vt, ((0, 0), (0, 0), (0, VPAD - V)))
  vmat = vt.reshape(B, TR, LANES)

  pe = HPAD - H
  a = rows[:H].astype(jnp.int32)
  bb = cols[:H].astype(jnp.int32)
  epack = jnp.pad(a * 32768 + bb, (0, pe))
  evala = jnp.pad(vals[:H].astype(jnp.float32), (0, pe))
  evalb = jnp.pad(vals[H:].astype(jnp.float32), (0, pe))

  out = _sc_laplacian(vmat, epack, evala, evalb, TR=TR, EW=EW, VPAD=VPAD)
  return jnp.sum(out) / (B * V)
